# baseline (device time: 85285 ns/iter reference)
import jax
import jax.numpy as jnp
from jax import lax
from jax.experimental import pallas as pl
from jax.experimental.pallas import tpu as pltpu


def kernel(ids, E):
    v_local, d = E.shape
    (t,) = ids.shape

    my_x = lax.axis_index("x")

    local = ids - my_x * v_local
    in_range = (local >= 0) & (local < v_local)
    safe = jnp.where(in_range, local, 0)
    partial = jnp.where(in_range[:, None], E[safe, :], 0.0).astype(jnp.float32)

    def body(p_ref, out_ref, comm_ref, send_sem, recv_sem):
        x = lax.axis_index("x")
        y = lax.axis_index("y")
        z = lax.axis_index("z")
        partner = (1 - x, y, z)

        barrier_sem = pltpu.get_barrier_semaphore()
        pl.semaphore_signal(
            barrier_sem, inc=1,
            device_id=partner, device_id_type=pl.DeviceIdType.MESH,
        )
        pl.semaphore_wait(barrier_sem, 1)

        rdma = pltpu.make_async_remote_copy(
            src_ref=p_ref,
            dst_ref=comm_ref,
            send_sem=send_sem,
            recv_sem=recv_sem,
            device_id=partner,
            device_id_type=pl.DeviceIdType.MESH,
        )
        rdma.start()
        rdma.wait()

        out_ref[...] = p_ref[...] + comm_ref[...]

    return pl.pallas_call(
        body,
        out_shape=jax.ShapeDtypeStruct((t, d), jnp.float32),
        in_specs=[pl.BlockSpec(memory_space=pltpu.VMEM)],
        out_specs=pl.BlockSpec(memory_space=pltpu.VMEM),
        scratch_shapes=[
            pltpu.VMEM((t, d), jnp.float32),
            pltpu.SemaphoreType.DMA,
            pltpu.SemaphoreType.DMA,
        ],
        compiler_params=pltpu.CompilerParams(collective_id=0),
    )(partial)


# device time: 44569 ns/iter; 1.9135x vs baseline; 1.9135x over previous
import jax
import jax.numpy as jnp
from jax import lax
from jax.experimental import pallas as pl
from jax.experimental.pallas import tpu as pltpu


def kernel(ids, E):
    v_local, d = E.shape
    (t_total,) = ids.shape

    def body(ids_ref, e_ref, out_ref, copy_sem, send_sem, recv_sem):
        x = lax.axis_index("x")
        y = lax.axis_index("y")
        z = lax.axis_index("z")
        partner = (1 - x, y, z)

        barrier_sem = pltpu.get_barrier_semaphore()
        pl.semaphore_signal(
            barrier_sem, inc=1,
            device_id=partner, device_id_type=pl.DeviceIdType.MESH,
        )
        pl.semaphore_wait(barrier_sem, 1)

        lo = x * v_local

        def step(t, k):
            row = ids_ref[t] - lo
            mine = (row >= 0) & (row < v_local)

            @pl.when(mine)
            def _():
                src = e_ref.at[pl.ds(row, 1), :]
                pltpu.make_async_copy(
                    src, out_ref.at[pl.ds(t, 1), :], copy_sem
                ).start()
                pltpu.make_async_remote_copy(
                    src_ref=src,
                    dst_ref=out_ref.at[pl.ds(t, 1), :],
                    send_sem=send_sem,
                    recv_sem=recv_sem,
                    device_id=partner,
                    device_id_type=pl.DeviceIdType.MESH,
                ).start()

            return k + jnp.where(mine, 1, 0).astype(jnp.int32)

        k = lax.fori_loop(0, t_total, step, jnp.int32(0))

        def drain_copy(_, c):
            pltpu.make_async_copy(
                e_ref.at[pl.ds(0, 1), :], out_ref.at[pl.ds(0, 1), :], copy_sem
            ).wait()
            return c

        def drain_send(_, c):
            pltpu.make_async_remote_copy(
                src_ref=e_ref.at[pl.ds(0, 1), :],
                dst_ref=out_ref.at[pl.ds(0, 1), :],
                send_sem=send_sem,
                recv_sem=recv_sem,
                device_id=partner,
                device_id_type=pl.DeviceIdType.MESH,
            ).wait_send()
            return c

        def drain_recv(_, c):
            pltpu.make_async_remote_copy(
                src_ref=e_ref.at[pl.ds(0, 1), :],
                dst_ref=out_ref.at[pl.ds(0, 1), :],
                send_sem=send_sem,
                recv_sem=recv_sem,
                device_id=partner,
                device_id_type=pl.DeviceIdType.MESH,
            ).wait_recv()
            return c

        lax.fori_loop(0, k, drain_copy, 0)
        lax.fori_loop(0, k, drain_send, 0)
        lax.fori_loop(0, t_total - k, drain_recv, 0)

    return pl.pallas_call(
        body,
        out_shape=jax.ShapeDtypeStruct((t_total, d), jnp.float32),
        in_specs=[
            pl.BlockSpec(memory_space=pltpu.SMEM),
            pl.BlockSpec(memory_space=pl.ANY),
        ],
        out_specs=pl.BlockSpec(memory_space=pltpu.VMEM),
        scratch_shapes=[
            pltpu.SemaphoreType.DMA,
            pltpu.SemaphoreType.DMA,
            pltpu.SemaphoreType.DMA,
        ],
        compiler_params=pltpu.CompilerParams(collective_id=0),
    )(ids, E)
